# baseline (speedup 1.0000x reference)
import functools

import jax
import jax.numpy as jnp
from jax import lax
from jax.experimental import pallas as pl
from jax.experimental.pallas import tpu as pltpu
from jax.experimental.pallas import tpu_sc as plsc

_B, _C, _H, _W, _K = 8, 3, 512, 512, 21
_BETA = 1.0
_HW = _H * _W
_NC, _NS, _L = 2, 16, 16
_NW = _NC * _NS
_KP = 24
_ACC_ROWS = 2 * _KP
_ACC_N = _ACC_ROWS * _L
_RCH = 16
_CH = _RCH * _W
_RSUB = _H // 4
_NCHUNK = _RSUB // _RCH
_JGRP = _W // _L
_NSLOT = 2


def _sc_body(inp_hbm, tgt_hbm, msk_hbm, out_hbm,
             i0, i1, t0, t1, m0, m1, acc, sem):
    ibufs = (i0, i1)
    tbufs = (t0, t1)
    mbufs = (m0, m1)

    cc = lax.axis_index("c")
    ss = lax.axis_index("s")
    wid = cc * _NS + ss
    b = wid // 4
    r0 = (wid % 4) * _RSUB

    zero = jnp.zeros((_L,), jnp.float32)
    for k in range(_ACC_ROWS):
        acc[pl.ds(k * _L, _L)] = zero

    def _copies(g, sl):
        h0 = pl.multiple_of(r0 + g * _RCH, _RCH)
        return [
            pltpu.make_async_copy(
                inp_hbm.at[b, :, pl.ds(h0, _RCH), :], ibufs[sl], sem),
            pltpu.make_async_copy(
                tgt_hbm.at[b, :, pl.ds(h0, _RCH), :], tbufs[sl], sem),
            pltpu.make_async_copy(
                msk_hbm.at[b, pl.ds(h0, _RCH), :], mbufs[sl], sem),
        ]

    def _start(g, sl):
        for cp in _copies(g, sl):
            cp.start()

    def _wait(g, sl):
        for cp in _copies(g, sl):
            cp.wait()

    lanes = lax.broadcasted_iota(jnp.int32, (_L,), 0)
    ones = jnp.ones((_L,), jnp.float32)

    def _compute(sl):
        ibc, tbc, mbc = ibufs[sl], tbufs[sl], mbufs[sl]

        @plsc.parallel_loop(0, _RCH, unroll=1)
        def _row(i):
            for jj in range(_JGRP):
                j = jj * _L
                d = (jnp.abs(ibc[0, i, pl.ds(j, _L)] - tbc[0, i, pl.ds(j, _L)])
                     + jnp.abs(ibc[1, i, pl.ds(j, _L)] - tbc[1, i, pl.ds(j, _L)])
                     + jnp.abs(ibc[2, i, pl.ds(j, _L)] - tbc[2, i, pl.ds(j, _L)]))
                mv = mbc[i, pl.ds(j, _L)]
                idx = mv * _L + lanes
                plsc.addupdate_scatter(acc, [idx], d)
                plsc.addupdate_scatter(acc, [idx + _KP * _L], ones)

    _start(0, 0)

    def _pair(p, carry):
        g0 = p * 2
        _start(g0 + 1, 1)
        _wait(g0, 0)
        _compute(0)

        @pl.when(g0 + 2 < _NCHUNK)
        def _():
            _start(g0 + 2, 0)

        _wait(g0 + 1, 1)
        _compute(1)
        return carry

    lax.fori_loop(0, _NCHUNK // 2, _pair, 0)

    pltpu.sync_copy(acc, out_hbm.at[pl.ds(wid * _ACC_N, _ACC_N)])


_sc_segment_sums = functools.partial(
    pl.kernel,
    mesh=plsc.VectorSubcoreMesh(core_axis_name="c", subcore_axis_name="s"),
    out_type=jax.ShapeDtypeStruct((_NW * _ACC_N,), jnp.float32),
    compiler_params=pltpu.CompilerParams(needs_layout_passes=False),
    scratch_types=(
        [pltpu.VMEM((_C, _RCH, _W), jnp.float32) for _ in range(2 * _NSLOT)]
        + [pltpu.VMEM((_RCH, _W), jnp.int32) for _ in range(_NSLOT)]
        + [pltpu.VMEM((_ACC_N,), jnp.float32), pltpu.SemaphoreType.DMA]
    ),
)(_sc_body)


def kernel(input, target, mask):
    part = _sc_segment_sums(input, target, mask)
    part = part.reshape(_B, _NW // _B, _ACC_ROWS, _L).sum(axis=(1, 3))
    sums = part[:, :_K]
    cnts = part[:, _KP:_KP + _K]
    avg = sums / jnp.maximum(cnts * _C, 1.0)
    m = avg.max()
    t = (avg * sums).sum()
    total = sums.sum()
    n = _B * _C * _H * _W
    return (total + _BETA * (t / m)) / n


# speedup vs baseline: 1.2054x; 1.2054x over previous
import functools

import jax
import jax.numpy as jnp
from jax import lax
from jax.experimental import pallas as pl
from jax.experimental.pallas import tpu as pltpu
from jax.experimental.pallas import tpu_sc as plsc

_B, _C, _H, _W, _K = 8, 3, 512, 512, 21
_BETA = 1.0
_HW = _H * _W
_NC, _NS, _L = 2, 16, 16
_NW = _NC * _NS
_KP = 24
_ACC_ROWS = 2 * _KP
_ACC_N = _ACC_ROWS * _L
_RT = 128
_RCH = 16
_CH = _RCH * _W
_RSUB = (_H - _RT) // 4
_NCHUNK = _RSUB // _RCH
_JGRP = _W // _L
_NSLOT = 2
_BH = 64


def _sc_body(inp_hbm, tgt_hbm, msk_hbm, out_hbm,
             i0, i1, t0, t1, m0, m1, acc, sem):
    ibufs = (i0, i1)
    tbufs = (t0, t1)
    mbufs = (m0, m1)

    cc = lax.axis_index("c")
    ss = lax.axis_index("s")
    wid = cc * _NS + ss
    b = wid // 4
    r0 = _RT + (wid % 4) * _RSUB

    zero = jnp.zeros((_L,), jnp.float32)
    for k in range(_ACC_ROWS):
        acc[pl.ds(k * _L, _L)] = zero

    def _copies(g, sl):
        h0 = pl.multiple_of(r0 + g * _RCH, _RCH)
        return [
            pltpu.make_async_copy(
                inp_hbm.at[b, :, pl.ds(h0, _RCH), :], ibufs[sl], sem),
            pltpu.make_async_copy(
                tgt_hbm.at[b, :, pl.ds(h0, _RCH), :], tbufs[sl], sem),
            pltpu.make_async_copy(
                msk_hbm.at[b, pl.ds(h0, _RCH), :], mbufs[sl], sem),
        ]

    def _start(g, sl):
        for cp in _copies(g, sl):
            cp.start()

    def _wait(g, sl):
        for cp in _copies(g, sl):
            cp.wait()

    lanes = lax.broadcasted_iota(jnp.int32, (_L,), 0)
    ones = jnp.ones((_L,), jnp.float32)

    _start(0, 0)
    for g in range(_NCHUNK):
        sl = g % _NSLOT
        if g + 1 < _NCHUNK:
            _start(g + 1, 1 - sl)
        _wait(g, sl)

        ibc, tbc, mbc = ibufs[sl], tbufs[sl], mbufs[sl]

        @plsc.parallel_loop(0, _RCH * _JGRP, unroll=8)
        def _it(t):
            i = lax.shift_right_logical(t, 5)
            j = lax.shift_left(lax.bitwise_and(t, _JGRP - 1), 4)
            d = (jnp.abs(ibc[0, i, pl.ds(j, _L)] - tbc[0, i, pl.ds(j, _L)])
                 + jnp.abs(ibc[1, i, pl.ds(j, _L)] - tbc[1, i, pl.ds(j, _L)])
                 + jnp.abs(ibc[2, i, pl.ds(j, _L)] - tbc[2, i, pl.ds(j, _L)]))
            mv = mbc[i, pl.ds(j, _L)]
            idx = mv * _L + lanes
            plsc.addupdate_scatter(acc, [idx], d)
            plsc.addupdate_scatter(acc, [idx + _KP * _L], ones)

    pltpu.sync_copy(acc, out_hbm.at[pl.ds(wid * _ACC_N, _ACC_N)])


_sc_segment_sums = functools.partial(
    pl.kernel,
    mesh=plsc.VectorSubcoreMesh(core_axis_name="c", subcore_axis_name="s"),
    out_type=jax.ShapeDtypeStruct((_NW * _ACC_N,), jnp.float32),
    compiler_params=pltpu.CompilerParams(needs_layout_passes=False),
    scratch_types=(
        [pltpu.VMEM((_C, _RCH, _W), jnp.float32) for _ in range(2 * _NSLOT)]
        + [pltpu.VMEM((_RCH, _W), jnp.int32) for _ in range(_NSLOT)]
        + [pltpu.VMEM((_ACC_N,), jnp.float32), pltpu.SemaphoreType.DMA]
    ),
)(_sc_body)


def _tc_body(inp_ref, tgt_ref, msk_ref, sums_ref, cnts_ref):
    bi = pl.program_id(0)
    hi = pl.program_id(1)

    @pl.when(jnp.logical_and(bi == 0, hi == 0))
    def _():
        sums_ref[...] = jnp.zeros_like(sums_ref)
        cnts_ref[...] = jnp.zeros_like(cnts_ref)

    d = jnp.sum(jnp.abs(inp_ref[0] - tgt_ref[0]), axis=0)
    m = msk_ref[0]
    svals, cvals = [], []
    for k in range(_K):
        sel = m == k
        svals.append(jnp.sum(jnp.where(sel, d, 0.0)))
        cvals.append(jnp.sum(sel.astype(jnp.float32)))
    pad = jnp.zeros((128 - _K,), jnp.float32)
    svec = jnp.concatenate([jnp.stack(svals), pad]).reshape(1, 128)
    cvec = jnp.concatenate([jnp.stack(cvals), pad]).reshape(1, 128)
    row = (lax.broadcasted_iota(jnp.int32, (_B, 1), 0) == bi).astype(
        jnp.float32)
    sums_ref[...] += row * svec
    cnts_ref[...] += row * cvec


_tc_partial = pl.pallas_call(
    _tc_body,
    grid=(_B, _RT // _BH),
    in_specs=[
        pl.BlockSpec((1, _C, _BH, _W), lambda b, h: (b, 0, h, 0)),
        pl.BlockSpec((1, _C, _BH, _W), lambda b, h: (b, 0, h, 0)),
        pl.BlockSpec((1, _BH, _W), lambda b, h: (b, h, 0)),
    ],
    out_specs=[
        pl.BlockSpec((_B, 128), lambda b, h: (0, 0)),
        pl.BlockSpec((_B, 128), lambda b, h: (0, 0)),
    ],
    out_shape=[
        jax.ShapeDtypeStruct((_B, 128), jnp.float32),
        jax.ShapeDtypeStruct((_B, 128), jnp.float32),
    ],
    compiler_params=pltpu.CompilerParams(
        dimension_semantics=("arbitrary", "arbitrary")),
)


def kernel(input, target, mask):
    part = _sc_segment_sums(input, target, mask)
    tc_sums, tc_cnts = _tc_partial(input, target, mask)
    part = part.reshape(_B, _NW // _B, _ACC_ROWS, _L).sum(axis=(1, 3))
    sums = part[:, :_K] + tc_sums[:, :_K]
    cnts = part[:, _KP:_KP + _K] + tc_cnts[:, :_K]
    avg = sums / jnp.maximum(cnts * _C, 1.0)
    m = avg.max()
    t = (avg * sums).sum()
    total = sums.sum()
    n = _B * _C * _H * _W
    return (total + _BETA * (t / m)) / n
